# baseline (speedup 1.0000x reference)
import functools

import jax
import jax.numpy as jnp
from jax import lax
from jax.experimental import pallas as pl
from jax.experimental.pallas import tpu as pltpu
from jax.experimental.pallas import tpu_sc as plsc

N = 10000
C1 = 128
C2 = 256
E = 320000
NP = 10240
EP = 327680
NSC = 2
NSUB = 16
CH = 80
CHUNKS = EP // (NSUB * CH)
CHUNKS_H = EP // (NSC * NSUB * CH)
WB = NP // NSUB
BLK = 1024


def _sc_mesh():
    return plsc.VectorSubcoreMesh(
        core_axis_name="c", subcore_axis_name="s",
        num_cores=NSC, num_subcores=NSUB)


def _hist_body(dst_hbm, zeros1, out, dst_v, ones_v, sem, acc):
    c = lax.axis_index("c")
    s = lax.axis_index("s")
    wid = c * NSUB + s
    for i in range(CH // 16):
        ones_v[pl.ds(i * 16, 16)] = jnp.full((16,), 1.0, jnp.float32)
    pltpu.async_copy(dst_hbm.at[pl.ds(wid * CHUNKS_H, CHUNKS_H)], dst_v, sem).wait()
    pltpu.sync_copy(zeros1.at[pl.ds(s * WB, WB)], acc.at[pl.ds(s * WB, WB)])
    plsc.subcore_barrier()

    def step(j, carry):
        pltpu.sync_copy(ones_v, acc.at[dst_v.at[j]], add=True)
        return carry

    lax.fori_loop(0, CHUNKS_H, step, 0)
    plsc.subcore_barrier()
    pltpu.sync_copy(acc.at[pl.ds(s * WB, WB)], out.at[c, pl.ds(s * WB, WB)])


def _hist(dst_r, zeros1):
    return pl.kernel(
        _hist_body,
        out_type=jax.ShapeDtypeStruct((NSC, NP), jnp.float32),
        mesh=_sc_mesh(),
        scratch_types=[
            pltpu.VMEM((CHUNKS_H, CH), jnp.int32),
            pltpu.VMEM((CH,), jnp.float32),
            pltpu.SemaphoreType.DMA,
            pltpu.VMEM_SHARED((NP,), jnp.float32),
        ],
    )(dst_r, zeros1)


def _agg_body(tab_hbm, src_hbm, dst_hbm, zeros2, out,
              src_v, db0, db1, gb0, gb1, sem, sem0, sem1, semd0, semd1, acc):
    c = lax.axis_index("c")
    s = lax.axis_index("s")

    @pl.when(c == 0)
    def _():
        chunk_base = s * CHUNKS

        pltpu.async_copy(src_hbm.at[pl.ds(chunk_base * CH, CHUNKS * CH)],
                         src_v, sem).wait()
        pltpu.sync_copy(zeros2.at[pl.ds(s * WB, WB)], acc.at[pl.ds(s * WB, WB)])
        plsc.subcore_barrier()

        def dload(j, db, semd):
            pltpu.async_copy(dst_hbm.at[pl.ds(chunk_base + j, 1)], db, semd)

        def gather(j, gb, semg):
            pltpu.async_copy(tab_hbm.at[src_v.at[pl.ds(j * CH, CH)]], gb, semg)

        dload(0, db0, semd0)
        dload(1, db1, semd1)
        gather(0, gb0, sem0)

        def step(i, carry):
            j0 = 2 * i
            j1 = 2 * i + 1
            gather(j1, gb1, sem1)
            pltpu.make_async_copy(tab_hbm.at[src_v.at[pl.ds(j0 * CH, CH)]],
                                  gb0, sem0).wait()
            pltpu.make_async_copy(dst_hbm.at[pl.ds(chunk_base + j0, 1)],
                                  db0, semd0).wait()
            pltpu.sync_copy(gb0, acc.at[db0.at[0]], add=True)

            @pl.when(i + 1 < CHUNKS // 2)
            def _():
                dload(j0 + 2, db0, semd0)
                gather(j0 + 2, gb0, sem0)

            pltpu.make_async_copy(tab_hbm.at[src_v.at[pl.ds(j1 * CH, CH)]],
                                  gb1, sem1).wait()
            pltpu.make_async_copy(dst_hbm.at[pl.ds(chunk_base + j1, 1)],
                                  db1, semd1).wait()
            pltpu.sync_copy(gb1, acc.at[db1.at[0]], add=True)

            @pl.when(i + 1 < CHUNKS // 2)
            def _():
                dload(j1 + 2, db1, semd1)

            return carry

        lax.fori_loop(0, CHUNKS // 2, step, 0)
        plsc.subcore_barrier()
        pltpu.sync_copy(acc.at[pl.ds(s * WB, WB)], out.at[pl.ds(s * WB, WB)])


def _agg(tab, src_f, dst_r, zeros2):
    return pl.kernel(
        _agg_body,
        out_type=jax.ShapeDtypeStruct((NP, C1), jnp.float32),
        mesh=_sc_mesh(),
        scratch_types=[
            pltpu.VMEM((CHUNKS * CH,), jnp.int32),
            pltpu.VMEM((1, CH), jnp.int32),
            pltpu.VMEM((1, CH), jnp.int32),
            pltpu.VMEM((CH, C1), jnp.float32),
            pltpu.VMEM((CH, C1), jnp.float32),
            pltpu.SemaphoreType.DMA,
            pltpu.SemaphoreType.DMA,
            pltpu.SemaphoreType.DMA,
            pltpu.SemaphoreType.DMA,
            pltpu.SemaphoreType.DMA,
            pltpu.VMEM_SHARED((NP, C1), jnp.float32),
        ],
    )(tab, src_f, dst_r, zeros2)


def _scale_body(h0, h1, x, xs_out, dis_out):
    i = pl.program_id(0)
    deg = h0[...] + h1[...] + 1.0
    d = lax.rsqrt(deg)
    rows = lax.broadcasted_iota(jnp.int32, (BLK, 1), 0) + i * BLK
    d = jnp.where(rows < N, d, 0.0)
    dis_out[...] = d
    xs_out[...] = x[...] * d


def _scale(h0, h1, x_p):
    return pl.pallas_call(
        _scale_body,
        grid=(NP // BLK,),
        in_specs=[
            pl.BlockSpec((BLK, 1), lambda i: (i, 0)),
            pl.BlockSpec((BLK, 1), lambda i: (i, 0)),
            pl.BlockSpec((BLK, C1), lambda i: (i, 0)),
        ],
        out_specs=[
            pl.BlockSpec((BLK, C1), lambda i: (i, 0)),
            pl.BlockSpec((BLK, 1), lambda i: (i, 0)),
        ],
        out_shape=[
            jax.ShapeDtypeStruct((NP, C1), jnp.float32),
            jax.ShapeDtypeStruct((NP, 1), jnp.float32),
        ],
    )(h0, h1, x_p)


def _mlp_body(a0, xs, dis, W1, b1, W2, g_out):
    a = (a0[...] + xs[...]) * dis[...]
    h = jnp.dot(a, W1[...], preferred_element_type=jnp.float32) + b1[...]
    h = jnp.maximum(h, 0.0) * dis[...]
    g_out[...] = jnp.dot(h, W2[...], preferred_element_type=jnp.float32)


def _mlp(a0, xs, dis, W1, b1, W2):
    return pl.pallas_call(
        _mlp_body,
        grid=(NP // BLK,),
        in_specs=[
            pl.BlockSpec((BLK, C1), lambda i: (i, 0)),
            pl.BlockSpec((BLK, C1), lambda i: (i, 0)),
            pl.BlockSpec((BLK, 1), lambda i: (i, 0)),
            pl.BlockSpec((C1, C2), lambda i: (0, 0)),
            pl.BlockSpec((1, C2), lambda i: (0, 0)),
            pl.BlockSpec((C2, C1), lambda i: (0, 0)),
        ],
        out_specs=pl.BlockSpec((BLK, C1), lambda i: (i, 0)),
        out_shape=jax.ShapeDtypeStruct((NP, C1), jnp.float32),
    )(a0, xs, dis, W1, b1, W2)


def _final_body(a0, g, dis, b2, out):
    out[...] = (a0[...] + g[...]) * dis[...] + b2[...]


def _final(a0, g, dis, b2):
    return pl.pallas_call(
        _final_body,
        grid=(NP // BLK,),
        in_specs=[
            pl.BlockSpec((BLK, C1), lambda i: (i, 0)),
            pl.BlockSpec((BLK, C1), lambda i: (i, 0)),
            pl.BlockSpec((BLK, 1), lambda i: (i, 0)),
            pl.BlockSpec((1, C1), lambda i: (0, 0)),
        ],
        out_specs=pl.BlockSpec((BLK, C1), lambda i: (i, 0)),
        out_shape=jax.ShapeDtypeStruct((NP, C1), jnp.float32),
    )(a0, g, dis, b2)


def kernel(x, edge_index, W1, b1, W2, b2):
    ei = edge_index.astype(jnp.int32)
    pad = jnp.full((EP - E,), N, jnp.int32)
    src_f = jnp.concatenate([ei[0], pad])
    dst_r = jnp.concatenate([ei[1], pad]).reshape(EP // CH, CH)
    x_p = jnp.pad(x, ((0, NP - N), (0, 0)))
    zeros1 = jnp.zeros((NP,), jnp.float32)
    zeros2 = jnp.zeros((NP, C1), jnp.float32)

    hist = _hist(dst_r, zeros1)
    h0 = hist[0].reshape(NP, 1)
    h1 = hist[1].reshape(NP, 1)
    xs, dis = _scale(h0, h1, x_p)
    agg1 = _agg(xs, src_f, dst_r, zeros2)
    g = _mlp(agg1, xs, dis, W1, b1.reshape(1, C2), W2)
    agg2 = _agg(g, src_f, dst_r, zeros2)
    out_p = _final(agg2, g, dis, b2.reshape(1, C1))
    return out_p[:N]


# speedup vs baseline: 1.5899x; 1.5899x over previous
import jax
import jax.numpy as jnp
from jax import lax
from jax.experimental import pallas as pl
from jax.experimental.pallas import tpu as pltpu
from jax.experimental.pallas import tpu_sc as plsc

N = 10000
C1 = 128
C2 = 256
E = 320000
NP = 10240
EP = 327680
NSC = 2
NSUB = 16
CH = 80
CHUNKS = EP // (NSC * NSUB * CH)
A0 = 240
A1 = 2 * CHUNKS - A0
AMAX = max(A0, A1)
WB = NP // NSUB
BLK = 1024


def _sc_mesh():
    return plsc.VectorSubcoreMesh(
        core_axis_name="c", subcore_axis_name="s",
        num_cores=NSC, num_subcores=NSUB)


def _hist_body(dst_hbm, zeros1, out, dst_v, ones_v, sem, acc):
    c = lax.axis_index("c")
    s = lax.axis_index("s")
    wid = c * NSUB + s
    for i in range(CH // 16):
        ones_v[pl.ds(i * 16, 16)] = jnp.full((16,), 1.0, jnp.float32)
    pltpu.async_copy(dst_hbm.at[pl.ds(wid * CHUNKS, CHUNKS)],
                     dst_v, sem).wait()
    pltpu.sync_copy(zeros1.at[pl.ds(s * WB, WB)], acc.at[pl.ds(s * WB, WB)])
    plsc.subcore_barrier()

    def step(j, carry):
        pltpu.sync_copy(ones_v, acc.at[dst_v.at[j]], add=True)
        return carry

    lax.fori_loop(0, CHUNKS, step, 0)
    plsc.subcore_barrier()
    pltpu.sync_copy(acc.at[pl.ds(s * WB, WB)], out.at[c, pl.ds(s * WB, WB)])


def _hist(dst_r, zeros1):
    return pl.kernel(
        _hist_body,
        out_type=jax.ShapeDtypeStruct((NSC, NP), jnp.float32),
        mesh=_sc_mesh(),
        scratch_types=[
            pltpu.VMEM((CHUNKS, CH), jnp.int32),
            pltpu.VMEM((CH,), jnp.float32),
            pltpu.SemaphoreType.DMA,
            pltpu.VMEM_SHARED((NP,), jnp.float32),
        ],
    )(dst_r, zeros1)


def _agg_body(tab_hbm, src_hbm, dst_hbm, zeros2, out,
              src_v, db0, db1, gb0, gb1, sem, sem0, sem1, semd0, semd1, acc):
    c = lax.axis_index("c")
    s = lax.axis_index("s")
    nchunks = jnp.where(c == 0, A0, A1)
    chunk_base = jnp.where(c == 0, s * A0, NSUB * A0 + s * A1)
    half = nchunks // 2

    pltpu.async_copy(src_hbm.at[pl.ds(chunk_base * CH, AMAX * CH)],
                     src_v, sem).wait()
    pltpu.sync_copy(zeros2.at[pl.ds(s * WB, WB)], acc.at[pl.ds(s * WB, WB)])
    plsc.subcore_barrier()

    def dload(j, db, semd):
        pltpu.async_copy(dst_hbm.at[pl.ds(chunk_base + j, 1)], db, semd)

    def gather(j, gb, semg):
        pltpu.async_copy(tab_hbm.at[src_v.at[pl.ds(j * CH, CH)]], gb, semg)

    dload(0, db0, semd0)
    dload(1, db1, semd1)
    gather(0, gb0, sem0)

    def step(i, carry):
        j0 = 2 * i
        j1 = 2 * i + 1
        gather(j1, gb1, sem1)
        pltpu.make_async_copy(tab_hbm.at[src_v.at[pl.ds(j0 * CH, CH)]],
                              gb0, sem0).wait()
        pltpu.make_async_copy(dst_hbm.at[pl.ds(chunk_base + j0, 1)],
                              db0, semd0).wait()
        pltpu.sync_copy(gb0, acc.at[db0.at[0]], add=True)

        @pl.when(i + 1 < half)
        def _():
            dload(j0 + 2, db0, semd0)
            gather(j0 + 2, gb0, sem0)

        pltpu.make_async_copy(tab_hbm.at[src_v.at[pl.ds(j1 * CH, CH)]],
                              gb1, sem1).wait()
        pltpu.make_async_copy(dst_hbm.at[pl.ds(chunk_base + j1, 1)],
                              db1, semd1).wait()
        pltpu.sync_copy(gb1, acc.at[db1.at[0]], add=True)

        @pl.when(i + 1 < half)
        def _():
            dload(j1 + 2, db1, semd1)

        return carry

    lax.fori_loop(0, half, step, 0)
    plsc.subcore_barrier()
    pltpu.sync_copy(acc.at[pl.ds(s * WB, WB)], out.at[c, pl.ds(s * WB, WB)])


def _agg(tab, src_f, dst_r, zeros2):
    return pl.kernel(
        _agg_body,
        out_type=jax.ShapeDtypeStruct((NSC, NP, C1), jnp.float32),
        mesh=_sc_mesh(),
        scratch_types=[
            pltpu.VMEM((AMAX * CH,), jnp.int32),
            pltpu.VMEM((1, CH), jnp.int32),
            pltpu.VMEM((1, CH), jnp.int32),
            pltpu.VMEM((CH, C1), jnp.float32),
            pltpu.VMEM((CH, C1), jnp.float32),
            pltpu.SemaphoreType.DMA,
            pltpu.SemaphoreType.DMA,
            pltpu.SemaphoreType.DMA,
            pltpu.SemaphoreType.DMA,
            pltpu.SemaphoreType.DMA,
            pltpu.VMEM_SHARED((NP, C1), jnp.float32),
        ],
    )(tab, src_f, dst_r, zeros2)


def _scale_body(h0, h1, x, xs_out, dis_out):
    i = pl.program_id(0)
    deg = h0[...] + h1[...] + 1.0
    d = lax.rsqrt(deg)
    rows = lax.broadcasted_iota(jnp.int32, (BLK, 1), 0) + i * BLK
    d = jnp.where(rows < N, d, 0.0)
    dis_out[...] = d
    xs_out[...] = x[...] * d


def _scale(h0, h1, x_p):
    return pl.pallas_call(
        _scale_body,
        grid=(NP // BLK,),
        in_specs=[
            pl.BlockSpec((BLK, 1), lambda i: (i, 0)),
            pl.BlockSpec((BLK, 1), lambda i: (i, 0)),
            pl.BlockSpec((BLK, C1), lambda i: (i, 0)),
        ],
        out_specs=[
            pl.BlockSpec((BLK, C1), lambda i: (i, 0)),
            pl.BlockSpec((BLK, 1), lambda i: (i, 0)),
        ],
        out_shape=[
            jax.ShapeDtypeStruct((NP, C1), jnp.float32),
            jax.ShapeDtypeStruct((NP, 1), jnp.float32),
        ],
    )(h0, h1, x_p)


def _mlp_body(a0, a1, xs, dis, W1, b1, W2, g_out):
    a = (a0[...] + a1[...] + xs[...]) * dis[...]
    h = jnp.dot(a, W1[...], preferred_element_type=jnp.float32) + b1[...]
    h = jnp.maximum(h, 0.0) * dis[...]
    g_out[...] = jnp.dot(h, W2[...], preferred_element_type=jnp.float32)


def _mlp(a0, a1, xs, dis, W1, b1, W2):
    return pl.pallas_call(
        _mlp_body,
        grid=(NP // BLK,),
        in_specs=[
            pl.BlockSpec((BLK, C1), lambda i: (i, 0)),
            pl.BlockSpec((BLK, C1), lambda i: (i, 0)),
            pl.BlockSpec((BLK, C1), lambda i: (i, 0)),
            pl.BlockSpec((BLK, 1), lambda i: (i, 0)),
            pl.BlockSpec((C1, C2), lambda i: (0, 0)),
            pl.BlockSpec((1, C2), lambda i: (0, 0)),
            pl.BlockSpec((C2, C1), lambda i: (0, 0)),
        ],
        out_specs=pl.BlockSpec((BLK, C1), lambda i: (i, 0)),
        out_shape=jax.ShapeDtypeStruct((NP, C1), jnp.float32),
    )(a0, a1, xs, dis, W1, b1, W2)


def _final_body(a0, a1, g, dis, b2, out):
    out[...] = (a0[...] + a1[...] + g[...]) * dis[...] + b2[...]


def _final(a0, a1, g, dis, b2):
    return pl.pallas_call(
        _final_body,
        grid=(NP // BLK,),
        in_specs=[
            pl.BlockSpec((BLK, C1), lambda i: (i, 0)),
            pl.BlockSpec((BLK, C1), lambda i: (i, 0)),
            pl.BlockSpec((BLK, C1), lambda i: (i, 0)),
            pl.BlockSpec((BLK, 1), lambda i: (i, 0)),
            pl.BlockSpec((1, C1), lambda i: (0, 0)),
        ],
        out_specs=pl.BlockSpec((BLK, C1), lambda i: (i, 0)),
        out_shape=jax.ShapeDtypeStruct((NP, C1), jnp.float32),
    )(a0, a1, g, dis, b2)


def kernel(x, edge_index, W1, b1, W2, b2):
    ei = edge_index.astype(jnp.int32)
    pad = jnp.full((EP - E,), N, jnp.int32)
    tail = jnp.full((AMAX * CH,), N, jnp.int32)
    src_f = jnp.concatenate([ei[0], pad, tail])
    dst_r = jnp.concatenate([ei[1], pad]).reshape(EP // CH, CH)
    x_p = jnp.pad(x, ((0, NP - N), (0, 0)))
    zeros1 = jnp.zeros((NP,), jnp.float32)
    zeros2 = jnp.zeros((NP, C1), jnp.float32)

    hist = _hist(dst_r, zeros1)
    h0 = hist[0].reshape(NP, 1)
    h1 = hist[1].reshape(NP, 1)
    xs, dis = _scale(h0, h1, x_p)
    agg1 = _agg(xs, src_f, dst_r, zeros2)
    g = _mlp(agg1[0], agg1[1], xs, dis, W1,
             b1.reshape(1, C2), W2)
    agg2 = _agg(g, src_f, dst_r, zeros2)
    out_p = _final(agg2[0], agg2[1], g, dis,
                   b2.reshape(1, C1))
    return out_p[:N]


# speedup vs baseline: 1.5999x; 1.0063x over previous
import jax
import jax.numpy as jnp
from jax import lax
from jax.experimental import pallas as pl
from jax.experimental.pallas import tpu as pltpu
from jax.experimental.pallas import tpu_sc as plsc

N = 10000
C1 = 128
C2 = 256
E = 320000
NP = 10240
EP = 327680
NSC = 2
NSUB = 16
CH = 80
CHUNKS = EP // (NSC * NSUB * CH)
A0 = 240
A1 = 2 * CHUNKS - A0
AMAX = max(A0, A1)
WB = NP // NSUB
BLK = 1024


def _sc_mesh():
    return plsc.VectorSubcoreMesh(
        core_axis_name="c", subcore_axis_name="s",
        num_cores=NSC, num_subcores=NSUB)


def _hist_body(dst_hbm, zeros1, out, dst_v, ones_v, sem, acc):
    c = lax.axis_index("c")
    s = lax.axis_index("s")
    wid = c * NSUB + s
    for i in range(CH // 16):
        ones_v[pl.ds(i * 16, 16)] = jnp.full((16,), 1.0, jnp.float32)
    pltpu.async_copy(dst_hbm.at[pl.ds(wid * CHUNKS, CHUNKS)],
                     dst_v, sem).wait()
    pltpu.sync_copy(zeros1.at[pl.ds(s * WB, WB)], acc.at[pl.ds(s * WB, WB)])
    plsc.subcore_barrier()

    def step(j, carry):
        pltpu.sync_copy(ones_v, acc.at[dst_v.at[j]], add=True)
        return carry

    lax.fori_loop(0, CHUNKS, step, 0)
    plsc.subcore_barrier()
    pltpu.sync_copy(acc.at[pl.ds(s * WB, WB)], out.at[c, pl.ds(s * WB, WB)])


def _hist(dst_r, zeros1):
    return pl.kernel(
        _hist_body,
        out_type=jax.ShapeDtypeStruct((NSC, NP), jnp.float32),
        mesh=_sc_mesh(),
        scratch_types=[
            pltpu.VMEM((CHUNKS, CH), jnp.int32),
            pltpu.VMEM((CH,), jnp.float32),
            pltpu.SemaphoreType.DMA,
            pltpu.VMEM_SHARED((NP,), jnp.float32),
        ],
    )(dst_r, zeros1)


def _agg_body(tab_hbm, src_hbm, dst_hbm, out,
              src_v, db0, db1, gb0, gb1, sem, sem0, sem1, semd0, semd1, acc):
    c = lax.axis_index("c")
    s = lax.axis_index("s")
    nchunks = jnp.where(c == 0, A0, A1)
    chunk_base = jnp.where(c == 0, s * A0, NSUB * A0 + s * A1)
    half = nchunks // 2

    pltpu.async_copy(src_hbm.at[pl.ds(chunk_base * CH, AMAX * CH)],
                     src_v, sem).wait()
    z16 = jnp.zeros((16,), jnp.float32)
    for r in range(CH):
        for k in range(C1 // 16):
            gb0[r, pl.ds(k * 16, 16)] = z16
    for m in range(WB // CH):
        pltpu.sync_copy(gb0, acc.at[pl.ds(s * WB + m * CH, CH)])
    plsc.subcore_barrier()

    def dload(j, db, semd):
        pltpu.async_copy(dst_hbm.at[pl.ds(chunk_base + j, 1)], db, semd)

    def gather(j, gb, semg):
        pltpu.async_copy(tab_hbm.at[src_v.at[pl.ds(j * CH, CH)]], gb, semg)

    dload(0, db0, semd0)
    dload(1, db1, semd1)
    gather(0, gb0, sem0)

    def step(i, carry):
        j0 = 2 * i
        j1 = 2 * i + 1
        gather(j1, gb1, sem1)
        pltpu.make_async_copy(tab_hbm.at[src_v.at[pl.ds(j0 * CH, CH)]],
                              gb0, sem0).wait()
        pltpu.make_async_copy(dst_hbm.at[pl.ds(chunk_base + j0, 1)],
                              db0, semd0).wait()
        pltpu.sync_copy(gb0, acc.at[db0.at[0]], add=True)

        @pl.when(i + 1 < half)
        def _():
            dload(j0 + 2, db0, semd0)
            gather(j0 + 2, gb0, sem0)

        pltpu.make_async_copy(tab_hbm.at[src_v.at[pl.ds(j1 * CH, CH)]],
                              gb1, sem1).wait()
        pltpu.make_async_copy(dst_hbm.at[pl.ds(chunk_base + j1, 1)],
                              db1, semd1).wait()
        pltpu.sync_copy(gb1, acc.at[db1.at[0]], add=True)

        @pl.when(i + 1 < half)
        def _():
            dload(j1 + 2, db1, semd1)

        return carry

    lax.fori_loop(0, half, step, 0)
    plsc.subcore_barrier()
    pltpu.sync_copy(acc.at[pl.ds(s * WB, WB)], out.at[c, pl.ds(s * WB, WB)])


def _agg(tab, src_f, dst_r):
    return pl.kernel(
        _agg_body,
        out_type=jax.ShapeDtypeStruct((NSC, NP, C1), jnp.float32),
        mesh=_sc_mesh(),
        scratch_types=[
            pltpu.VMEM((AMAX * CH,), jnp.int32),
            pltpu.VMEM((1, CH), jnp.int32),
            pltpu.VMEM((1, CH), jnp.int32),
            pltpu.VMEM((CH, C1), jnp.float32),
            pltpu.VMEM((CH, C1), jnp.float32),
            pltpu.SemaphoreType.DMA,
            pltpu.SemaphoreType.DMA,
            pltpu.SemaphoreType.DMA,
            pltpu.SemaphoreType.DMA,
            pltpu.SemaphoreType.DMA,
            pltpu.VMEM_SHARED((NP, C1), jnp.float32),
        ],
    )(tab, src_f, dst_r)


def _scale_body(h0, h1, x, xs_out, dis_out):
    i = pl.program_id(0)
    deg = h0[...] + h1[...] + 1.0
    d = lax.rsqrt(deg)
    rows = lax.broadcasted_iota(jnp.int32, (BLK, 1), 0) + i * BLK
    d = jnp.where(rows < N, d, 0.0)
    dis_out[...] = d
    xs_out[...] = x[...] * d


def _scale(h0, h1, x_p):
    return pl.pallas_call(
        _scale_body,
        grid=(NP // BLK,),
        in_specs=[
            pl.BlockSpec((BLK, 1), lambda i: (i, 0)),
            pl.BlockSpec((BLK, 1), lambda i: (i, 0)),
            pl.BlockSpec((BLK, C1), lambda i: (i, 0)),
        ],
        out_specs=[
            pl.BlockSpec((BLK, C1), lambda i: (i, 0)),
            pl.BlockSpec((BLK, 1), lambda i: (i, 0)),
        ],
        out_shape=[
            jax.ShapeDtypeStruct((NP, C1), jnp.float32),
            jax.ShapeDtypeStruct((NP, 1), jnp.float32),
        ],
    )(h0, h1, x_p)


def _mlp_body(a0, a1, xs, dis, W1, b1, W2, g_out):
    a = (a0[...] + a1[...] + xs[...]) * dis[...]
    h = jnp.dot(a, W1[...], preferred_element_type=jnp.float32) + b1[...]
    h = jnp.maximum(h, 0.0) * dis[...]
    g_out[...] = jnp.dot(h, W2[...], preferred_element_type=jnp.float32)


def _mlp(a0, a1, xs, dis, W1, b1, W2):
    return pl.pallas_call(
        _mlp_body,
        grid=(NP // BLK,),
        in_specs=[
            pl.BlockSpec((BLK, C1), lambda i: (i, 0)),
            pl.BlockSpec((BLK, C1), lambda i: (i, 0)),
            pl.BlockSpec((BLK, C1), lambda i: (i, 0)),
            pl.BlockSpec((BLK, 1), lambda i: (i, 0)),
            pl.BlockSpec((C1, C2), lambda i: (0, 0)),
            pl.BlockSpec((1, C2), lambda i: (0, 0)),
            pl.BlockSpec((C2, C1), lambda i: (0, 0)),
        ],
        out_specs=pl.BlockSpec((BLK, C1), lambda i: (i, 0)),
        out_shape=jax.ShapeDtypeStruct((NP, C1), jnp.float32),
    )(a0, a1, xs, dis, W1, b1, W2)


def _final_body(a0, a1, g, dis, b2, out):
    out[...] = (a0[...] + a1[...] + g[...]) * dis[...] + b2[...]


def _final(a0, a1, g, dis, b2):
    return pl.pallas_call(
        _final_body,
        grid=(NP // BLK,),
        in_specs=[
            pl.BlockSpec((BLK, C1), lambda i: (i, 0)),
            pl.BlockSpec((BLK, C1), lambda i: (i, 0)),
            pl.BlockSpec((BLK, C1), lambda i: (i, 0)),
            pl.BlockSpec((BLK, 1), lambda i: (i, 0)),
            pl.BlockSpec((1, C1), lambda i: (0, 0)),
        ],
        out_specs=pl.BlockSpec((BLK, C1), lambda i: (i, 0)),
        out_shape=jax.ShapeDtypeStruct((NP, C1), jnp.float32),
    )(a0, a1, g, dis, b2)


def kernel(x, edge_index, W1, b1, W2, b2):
    ei = edge_index.astype(jnp.int32)
    pad = jnp.full((EP - E,), N, jnp.int32)
    tail = jnp.full((AMAX * CH,), N, jnp.int32)
    src_f = jnp.concatenate([ei[0], pad, tail])
    dst_r = jnp.concatenate([ei[1], pad]).reshape(EP // CH, CH)
    x_p = jnp.pad(x, ((0, NP - N), (0, 0)))
    zeros1 = jnp.zeros((NP,), jnp.float32)

    hist = _hist(dst_r, zeros1)
    h0 = hist[0].reshape(NP, 1)
    h1 = hist[1].reshape(NP, 1)
    xs, dis = _scale(h0, h1, x_p)
    agg1 = _agg(xs, src_f, dst_r)
    g = _mlp(agg1[0], agg1[1], xs, dis, W1,
             b1.reshape(1, C2), W2)
    agg2 = _agg(g, src_f, dst_r)
    out_p = _final(agg2[0], agg2[1], g, dis,
                   b2.reshape(1, C1))
    return out_p[:N]
